# baseline (speedup 1.0000x reference)
import functools

import jax
import jax.numpy as jnp
from jax.experimental import pallas as pl
from jax.experimental.pallas import tpu as pltpu

_EPS = 1e-8
_LAT = 128
_NS = 8
_NS_EGO = 4


def _ego_body(*refs):
    sim_slices = refs[:_NS_EGO]
    w_ref, b_ref, e_ref, n_ref = refs[_NS_EGO:]
    kw = sim_slices[0].shape[1]
    w = w_ref[...].astype(jnp.bfloat16)
    acc = jnp.zeros((sim_slices[0].shape[0], _LAT), jnp.float32)
    for s in range(_NS_EGO):
        acc += jax.lax.dot_general(
            sim_slices[s][...].astype(jnp.bfloat16),
            w[s * kw:(s + 1) * kw, :],
            (((1,), (0,)), ((), ())), preferred_element_type=jnp.float32)
    e = jax.nn.sigmoid(acc + b_ref[...])
    e_ref[...] = e
    n_ref[...] = 1.0 / jnp.maximum(
        jnp.sqrt(jnp.sum(e * e, axis=1, keepdims=True)), _EPS)


def _ego_call(sim, w, b, bm):
    r = sim.shape[0]
    kw = r // _NS_EGO
    sim_specs = [
        pl.BlockSpec((bm, kw), lambda i, s=s: (i, s)) for s in range(_NS_EGO)
    ]
    return pl.pallas_call(
        _ego_body,
        grid=(r // bm,),
        in_specs=sim_specs + [
            pl.BlockSpec((r, _LAT), lambda i: (0, 0)),
            pl.BlockSpec((1, _LAT), lambda i: (0, 0)),
        ],
        out_specs=[
            pl.BlockSpec((bm, _LAT), lambda i: (i, 0)),
            pl.BlockSpec((bm, 1), lambda i: (i, 0)),
        ],
        out_shape=[
            jax.ShapeDtypeStruct((r, _LAT), jnp.float32),
            jax.ShapeDtypeStruct((r, 1), jnp.float32),
        ],
    )(*([sim] * _NS_EGO), w, b.reshape(1, _LAT))


def _prep_body(*refs):
    a_slices = refs[:_NS]
    ego_ref, w_ref = refs[_NS:_NS + 2]
    ab_slices = refs[_NS + 2:2 * _NS + 2]
    d_ref, g_ref = refs[2 * _NS + 2:]
    dg = jnp.zeros((a_slices[0].shape[0], 1), jnp.float32)
    for s in range(_NS):
        blk = a_slices[s][...]
        ab_slices[s][...] = blk.astype(jnp.bfloat16)
        dg += jnp.sum(blk, axis=1, keepdims=True)
    d = jnp.where(dg > 0, jax.lax.rsqrt(jnp.maximum(dg, _EPS)), 0.0)
    d_ref[...] = d
    g_ref[...] = jax.lax.dot_general(
        (d * ego_ref[...]).astype(jnp.bfloat16),
        w_ref[...].astype(jnp.bfloat16),
        (((1,), (0,)), ((), ())),
        preferred_element_type=jnp.float32).astype(jnp.bfloat16)


def _prep_call(a, ego, w, bm):
    n = a.shape[0]
    kw = n // _NS
    a_specs = [
        pl.BlockSpec((bm, kw), lambda i, s=s: (i, s)) for s in range(_NS)
    ]
    ab_specs = [pl.BlockSpec((bm, kw), lambda i: (i, 0))] * _NS
    return pl.pallas_call(
        _prep_body,
        grid=(n // bm,),
        in_specs=a_specs + [
            pl.BlockSpec((bm, _LAT), lambda i: (i, 0)),
            pl.BlockSpec((_LAT, _LAT), lambda i: (0, 0)),
        ],
        out_specs=ab_specs + [
            pl.BlockSpec((bm, 1), lambda i: (i, 0)),
            pl.BlockSpec((bm, _LAT), lambda i: (i, 0)),
        ],
        out_shape=[jax.ShapeDtypeStruct((n, kw), jnp.bfloat16)] * _NS + [
            jax.ShapeDtypeStruct((n, 1), jnp.float32),
            jax.ShapeDtypeStruct((n, _LAT), jnp.bfloat16),
        ],
    )(*([a] * _NS), ego, w)


def _layer_body(*refs, ns):
    a_slices = refs[:ns]
    g_ref, d_ref, ego_ref, en_ref, w_ref, contrib_ref, gnext_ref = \
        refs[ns:ns + 7]
    acc2 = refs[ns + 7]
    kw = a_slices[0].shape[1]
    bm = a_slices[0].shape[0]
    i = pl.program_id(0)
    last = pl.num_programs(0) - 1
    p = jax.lax.rem(i, 2)

    @pl.when(i < last)
    def _mm():
        acc = jax.lax.dot_general(
            a_slices[0][...], g_ref[pl.ds(0, kw), :],
            (((1,), (0,)), ((), ())), preferred_element_type=jnp.float32)
        for s in range(1, ns):
            acc += jax.lax.dot_general(
                a_slices[s][...], g_ref[pl.ds(s * kw, kw), :],
                (((1,), (0,)), ((), ())), preferred_element_type=jnp.float32)
        acc2[pl.ds(p, 1), :, :] = acc[None]

    @pl.when(i > 0)
    def _epi():
        acc = acc2[pl.ds(1 - p, 1), :, :].reshape(bm, _LAT)
        d = d_ref[...]
        lay = jnp.maximum(d * acc, 0.0)
        ones = jnp.ones((_LAT, 1), jnp.float32)
        ln2 = jax.lax.dot_general(lay * lay, ones, (((1,), (0,)), ((), ())),
                                  preferred_element_type=jnp.float32)
        dt = jax.lax.dot_general(lay * ego_ref[...], ones,
                                 (((1,), (0,)), ((), ())),
                                 preferred_element_type=jnp.float32)
        wgt = dt * jax.lax.rsqrt(jnp.maximum(ln2, _EPS * _EPS)) * en_ref[...]
        contrib_ref[...] = wgt * lay
        gnext_ref[...] = jax.lax.dot_general(
            (d * lay).astype(jnp.bfloat16), w_ref[...].astype(jnp.bfloat16),
            (((1,), (0,)), ((), ())),
            preferred_element_type=jnp.float32).astype(jnp.bfloat16)


def _layer_call(a_slices, g, d, ego, en, w, bm):
    n = g.shape[0]
    ns = len(a_slices)
    kw = n // ns
    nblk = n // bm
    return pl.pallas_call(
        functools.partial(_layer_body, ns=ns),
        grid=(nblk + 1,),
        in_specs=[
            pl.BlockSpec((bm, kw), lambda i: (jnp.minimum(i, nblk - 1), 0))
        ] * ns + [
            pl.BlockSpec((n, _LAT), lambda i: (0, 0)),
            pl.BlockSpec((bm, 1), lambda i: (jnp.maximum(i - 1, 0), 0)),
            pl.BlockSpec((bm, _LAT), lambda i: (jnp.maximum(i - 1, 0), 0)),
            pl.BlockSpec((bm, 1), lambda i: (jnp.maximum(i - 1, 0), 0)),
            pl.BlockSpec((_LAT, _LAT), lambda i: (0, 0)),
        ],
        out_specs=[
            pl.BlockSpec((bm, _LAT), lambda i: (jnp.maximum(i - 1, 0), 0)),
            pl.BlockSpec((bm, _LAT), lambda i: (jnp.maximum(i - 1, 0), 0)),
        ],
        out_shape=[
            jax.ShapeDtypeStruct((n, _LAT), jnp.float32),
            jax.ShapeDtypeStruct((n, _LAT), jnp.bfloat16),
        ],
        scratch_shapes=[pltpu.VMEM((2, bm, _LAT), jnp.float32)],
    )(*a_slices, g, d, ego, en, w)


def _pred_body(l1, l2, l3, d1, d2, d3, out_ref):
    lm = ((l1[...] + l2[...] + l3[...]) * (1.0 / 3.0)).astype(jnp.bfloat16)
    dm = ((d1[...] + d2[...] + d3[...]) * (1.0 / 3.0)).astype(jnp.bfloat16)
    out_ref[...] = jax.lax.dot_general(
        lm, dm, (((1,), (1,)), ((), ())), preferred_element_type=jnp.float32)


def _pred_call(ls, ds, bm):
    lr = ls[0].shape[0]
    dr = ds[0].shape[0]
    return pl.pallas_call(
        _pred_body,
        grid=(lr // bm,),
        in_specs=[pl.BlockSpec((bm, _LAT), lambda i: (i, 0))] * 3
        + [pl.BlockSpec((dr, _LAT), lambda i: (0, 0))] * 3,
        out_specs=pl.BlockSpec((bm, dr), lambda i: (i, 0)),
        out_shape=jax.ShapeDtypeStruct((lr, dr), jnp.float32),
    )(*ls, *ds)


def kernel(A_stack, lnc_sim, dis_sim, miR_sim, W_l, b_l, W_d, b_d, W_m, b_m,
           weight):
    l_num = lnc_sim.shape[0]
    d_num = dis_sim.shape[0]
    n = A_stack.shape[0]

    e_l, n_l = _ego_call(lnc_sim, W_l, b_l, bm=min(512, l_num))
    e_d, n_d = _ego_call(dis_sim, W_d, b_d, bm=min(1024, d_num))
    e_m, n_m = _ego_call(miR_sim, W_m, b_m, bm=min(1024, d_num))
    ego_all = jnp.concatenate([e_l, e_d, e_m], axis=0)
    en = jnp.concatenate([n_l, n_d, n_m], axis=0)

    *a_slices, d, g = _prep_call(A_stack, ego_all, weight, bm=min(512, n))

    contribs = []
    for _ in range(3):
        contrib, g = _layer_call(a_slices, g, d, ego_all, en, weight,
                                 bm=min(512, n))
        contribs.append(contrib)

    ls = [c[:l_num] for c in contribs]
    ds = [c[l_num:l_num + d_num] for c in contribs]
    return _pred_call(ls, ds, bm=min(1024, l_num))


# speedup vs baseline: 1.0548x; 1.0548x over previous
import functools

import jax
import jax.numpy as jnp
from jax.experimental import pallas as pl
from jax.experimental.pallas import tpu as pltpu

_EPS = 1e-8
_LAT = 128
_NS = 8
_NS_EGO = 4


def _ego_body(*refs):
    sim_slices = refs[:_NS_EGO]
    w_ref, b_ref, e_ref, n_ref = refs[_NS_EGO:]
    kw = sim_slices[0].shape[1]
    w = w_ref[...].astype(jnp.bfloat16)
    acc = jnp.zeros((sim_slices[0].shape[0], _LAT), jnp.float32)
    for s in range(_NS_EGO):
        acc += jax.lax.dot_general(
            sim_slices[s][...].astype(jnp.bfloat16),
            w[s * kw:(s + 1) * kw, :],
            (((1,), (0,)), ((), ())), preferred_element_type=jnp.float32)
    e = jax.nn.sigmoid(acc + b_ref[...])
    e_ref[...] = e
    n_ref[...] = 1.0 / jnp.maximum(
        jnp.sqrt(jnp.sum(e * e, axis=1, keepdims=True)), _EPS)


def _ego_call(sim, w, b, bm):
    r = sim.shape[0]
    kw = r // _NS_EGO
    sim_specs = [
        pl.BlockSpec((bm, kw), lambda i, s=s: (i, s)) for s in range(_NS_EGO)
    ]
    return pl.pallas_call(
        _ego_body,
        grid=(r // bm,),
        in_specs=sim_specs + [
            pl.BlockSpec((r, _LAT), lambda i: (0, 0)),
            pl.BlockSpec((1, _LAT), lambda i: (0, 0)),
        ],
        out_specs=[
            pl.BlockSpec((bm, _LAT), lambda i: (i, 0)),
            pl.BlockSpec((bm, 1), lambda i: (i, 0)),
        ],
        out_shape=[
            jax.ShapeDtypeStruct((r, _LAT), jnp.float32),
            jax.ShapeDtypeStruct((r, 1), jnp.float32),
        ],
    )(*([sim] * _NS_EGO), w, b.reshape(1, _LAT))


def _prep_body(*refs):
    a_slices = refs[:_NS]
    ego_ref, w_ref = refs[_NS:_NS + 2]
    ab_slices = refs[_NS + 2:2 * _NS + 2]
    d_ref, g_ref = refs[2 * _NS + 2:]
    dg = jnp.zeros((a_slices[0].shape[0], 1), jnp.float32)
    for s in range(_NS):
        blk = a_slices[s][...]
        ab_slices[s][...] = blk.astype(jnp.bfloat16)
        dg += jnp.sum(blk, axis=1, keepdims=True)
    d = jnp.where(dg > 0, jax.lax.rsqrt(jnp.maximum(dg, _EPS)), 0.0)
    d_ref[...] = d
    g_ref[...] = jax.lax.dot_general(
        (d * ego_ref[...]).astype(jnp.bfloat16),
        w_ref[...].astype(jnp.bfloat16),
        (((1,), (0,)), ((), ())),
        preferred_element_type=jnp.float32).astype(jnp.bfloat16)


def _prep_call(a, ego, w, bm):
    n = a.shape[0]
    kw = n // _NS
    a_specs = [
        pl.BlockSpec((bm, kw), lambda i, s=s: (i, s)) for s in range(_NS)
    ]
    ab_specs = [pl.BlockSpec((bm, kw), lambda i: (i, 0))] * _NS
    return pl.pallas_call(
        _prep_body,
        grid=(n // bm,),
        in_specs=a_specs + [
            pl.BlockSpec((bm, _LAT), lambda i: (i, 0)),
            pl.BlockSpec((_LAT, _LAT), lambda i: (0, 0)),
        ],
        out_specs=ab_specs + [
            pl.BlockSpec((bm, 1), lambda i: (i, 0)),
            pl.BlockSpec((bm, _LAT), lambda i: (i, 0)),
        ],
        out_shape=[jax.ShapeDtypeStruct((n, kw), jnp.bfloat16)] * _NS + [
            jax.ShapeDtypeStruct((n, 1), jnp.float32),
            jax.ShapeDtypeStruct((n, _LAT), jnp.bfloat16),
        ],
    )(*([a] * _NS), ego, w)


def _layer_body(*refs, ns, with_gnext):
    a_slices = refs[:ns]
    if with_gnext:
        g_ref, d_ref, ego_ref, en_ref, w_ref, contrib_ref, gnext_ref = \
            refs[ns:ns + 7]
        acc2 = refs[ns + 7]
    else:
        g_ref, d_ref, ego_ref, en_ref, w_ref, contrib_ref = refs[ns:ns + 6]
        acc2 = refs[ns + 6]
    kw = a_slices[0].shape[1]
    bm = a_slices[0].shape[0]
    i = pl.program_id(0)
    last = pl.num_programs(0) - 1
    p = jax.lax.rem(i, 2)

    @pl.when(i < last)
    def _mm():
        acc = jax.lax.dot_general(
            a_slices[0][...], g_ref[pl.ds(0, kw), :],
            (((1,), (0,)), ((), ())), preferred_element_type=jnp.float32)
        for s in range(1, ns):
            acc += jax.lax.dot_general(
                a_slices[s][...], g_ref[pl.ds(s * kw, kw), :],
                (((1,), (0,)), ((), ())), preferred_element_type=jnp.float32)
        acc2[pl.ds(p, 1), :, :] = acc[None]

    @pl.when(i > 0)
    def _epi():
        acc = acc2[pl.ds(1 - p, 1), :, :].reshape(bm, _LAT)
        d = d_ref[...]
        lay = jnp.maximum(d * acc, 0.0)
        ones = jnp.ones((_LAT, 1), jnp.float32)
        ln2 = jax.lax.dot_general(lay * lay, ones, (((1,), (0,)), ((), ())),
                                  preferred_element_type=jnp.float32)
        dt = jax.lax.dot_general(lay * ego_ref[...], ones,
                                 (((1,), (0,)), ((), ())),
                                 preferred_element_type=jnp.float32)
        wgt = dt * jax.lax.rsqrt(jnp.maximum(ln2, _EPS * _EPS)) * en_ref[...]
        contrib_ref[...] = wgt * lay
        if with_gnext:
            gnext_ref[...] = jax.lax.dot_general(
                (d * lay).astype(jnp.bfloat16),
                w_ref[...].astype(jnp.bfloat16),
                (((1,), (0,)), ((), ())),
                preferred_element_type=jnp.float32).astype(jnp.bfloat16)


def _layer_call(a_slices, g, d, ego, en, w, bm, rows=None, with_gnext=True):
    n = g.shape[0]
    if rows is None:
        rows = n
    ns = len(a_slices)
    kw = n // ns
    nblk = rows // bm
    out_specs = [pl.BlockSpec((bm, _LAT), lambda i: (jnp.maximum(i - 1, 0), 0))]
    out_shape = [jax.ShapeDtypeStruct((rows, _LAT), jnp.float32)]
    if with_gnext:
        out_specs.append(
            pl.BlockSpec((bm, _LAT), lambda i: (jnp.maximum(i - 1, 0), 0)))
        out_shape.append(jax.ShapeDtypeStruct((rows, _LAT), jnp.bfloat16))
    return pl.pallas_call(
        functools.partial(_layer_body, ns=ns, with_gnext=with_gnext),
        grid=(nblk + 1,),
        in_specs=[
            pl.BlockSpec((bm, kw), lambda i: (jnp.minimum(i, nblk - 1), 0))
        ] * ns + [
            pl.BlockSpec((n, _LAT), lambda i: (0, 0)),
            pl.BlockSpec((bm, 1), lambda i: (jnp.maximum(i - 1, 0), 0)),
            pl.BlockSpec((bm, _LAT), lambda i: (jnp.maximum(i - 1, 0), 0)),
            pl.BlockSpec((bm, 1), lambda i: (jnp.maximum(i - 1, 0), 0)),
            pl.BlockSpec((_LAT, _LAT), lambda i: (0, 0)),
        ],
        out_specs=out_specs,
        out_shape=out_shape,
        scratch_shapes=[pltpu.VMEM((2, bm, _LAT), jnp.float32)],
    )(*a_slices, g, d, ego, en, w)


def _pred_body(l1, l2, l3, d1, d2, d3, out_ref):
    lm = ((l1[...] + l2[...] + l3[...]) * (1.0 / 3.0)).astype(jnp.bfloat16)
    dm = ((d1[...] + d2[...] + d3[...]) * (1.0 / 3.0)).astype(jnp.bfloat16)
    out_ref[...] = jax.lax.dot_general(
        lm, dm, (((1,), (1,)), ((), ())), preferred_element_type=jnp.float32)


def _pred_call(ls, ds, bm):
    lr = ls[0].shape[0]
    dr = ds[0].shape[0]
    return pl.pallas_call(
        _pred_body,
        grid=(lr // bm,),
        in_specs=[pl.BlockSpec((bm, _LAT), lambda i: (i, 0))] * 3
        + [pl.BlockSpec((dr, _LAT), lambda i: (0, 0))] * 3,
        out_specs=pl.BlockSpec((bm, dr), lambda i: (i, 0)),
        out_shape=jax.ShapeDtypeStruct((lr, dr), jnp.float32),
    )(*ls, *ds)


def kernel(A_stack, lnc_sim, dis_sim, miR_sim, W_l, b_l, W_d, b_d, W_m, b_m,
           weight):
    l_num = lnc_sim.shape[0]
    d_num = dis_sim.shape[0]
    n = A_stack.shape[0]

    e_l, n_l = _ego_call(lnc_sim, W_l, b_l, bm=min(512, l_num))
    e_d, n_d = _ego_call(dis_sim, W_d, b_d, bm=min(1024, d_num))
    e_m, n_m = _ego_call(miR_sim, W_m, b_m, bm=min(1024, d_num))
    ego_all = jnp.concatenate([e_l, e_d, e_m], axis=0)
    en = jnp.concatenate([n_l, n_d, n_m], axis=0)

    *a_slices, d, g = _prep_call(A_stack, ego_all, weight, bm=min(512, n))

    contribs = []
    for _ in range(2):
        contrib, g = _layer_call(a_slices, g, d, ego_all, en, weight,
                                 bm=min(1024, n))
        contribs.append(contrib)
    ld = l_num + d_num
    bm3 = 1024 if ld % 1024 == 0 else 512
    contribs.append(_layer_call(a_slices, g, d, ego_all, en, weight,
                                bm=min(bm3, ld), rows=ld,
                                with_gnext=False)[0])

    ls = [c[:l_num] for c in contribs]
    ds = [c[l_num:l_num + d_num] for c in contribs]
    return _pred_call(ls, ds, bm=min(1024, l_num))
